# Initial kernel scaffold; baseline (speedup 1.0000x reference)
import functools

import jax
import jax.numpy as jnp
from jax import lax
from jax.experimental import pallas as pl
from jax.experimental.pallas import tpu as pltpu
from jax.experimental.pallas import tpu_sc as plsc

_LANES = 16
_IDX_PER_DMA = 128


@functools.lru_cache(maxsize=None)
def _make_kernel(B, L, V, D, NC, NS):
    NW = NC * NS
    bags_w = B // NW
    GROUP = 32
    idx_per_group = GROUP * L
    n_dma = idx_per_group // _IDX_PER_DMA
    n_groups = bags_w // GROUP
    CH = D // _LANES

    mesh = plsc.VectorSubcoreMesh(core_axis_name="c", subcore_axis_name="s")

    @functools.partial(
        pl.kernel,
        out_type=jax.ShapeDtypeStruct((B, 4 * D), jnp.float32),
        mesh=mesh,
        scratch_types=[
            pltpu.VMEM((bags_w * L,), jnp.int32),
            pltpu.VMEM((bags_w * L + _LANES,), jnp.float32),
            pltpu.VMEM((idx_per_group, D), jnp.float32),
            pltpu.VMEM((bags_w, 4 * D), jnp.float32),
            pltpu.SemaphoreType.DMA,
        ],
    )
    def k(i0, i1, wi0, wi1, w0, w1, t0, t1, wt0, wt1,
          out, idx_v, w_v, rows_v, out_v, sem):
        wid = lax.axis_index("s") * NC + lax.axis_index("c")
        base_idx = wid * (bags_w * L)
        tables = ((i0, None, t0, 0), (i1, None, t1, 1),
                  (wi0, w0, wt0, 2), (wi1, w1, wt1, 3))
        for ih, wh, th, t in tables:
            pltpu.sync_copy(ih.at[pl.ds(base_idx, bags_w * L)], idx_v)
            if wh is not None:
                pltpu.sync_copy(wh.at[pl.ds(base_idx, bags_w * L)],
                                w_v.at[pl.ds(0, bags_w * L)])

            def group_body(g, carry, wh=wh, th=th, t=t):
                copies = [
                    pltpu.async_copy(
                        th.at[idx_v.at[pl.ds(g * idx_per_group
                                             + kd * _IDX_PER_DMA,
                                             _IDX_PER_DMA)]],
                        rows_v.at[pl.ds(kd * _IDX_PER_DMA, _IDX_PER_DMA)],
                        sem)
                    for kd in range(n_dma)
                ]
                for cpy in copies:
                    cpy.wait()

                def bag_body(j, carry2):
                    row0 = j * L
                    bag = g * GROUP + j
                    if wh is not None:
                        wa = w_v[pl.ds(bag * L, _LANES)]
                        wb = w_v[pl.ds(bag * L + _LANES, _LANES)]
                    accs = [jnp.zeros((_LANES,), jnp.float32)
                            for _ in range(CH)]
                    for l in range(L):
                        if wh is not None:
                            src = wa if l < _LANES else wb
                            wl = jnp.take_along_axis(
                                src,
                                jnp.full((_LANES,), l % _LANES, jnp.int32),
                                axis=0)
                        for c in range(CH):
                            r = rows_v[row0 + l, pl.ds(c * _LANES, _LANES)]
                            accs[c] = accs[c] + (r * wl if wh is not None
                                                 else r)
                    for c in range(CH):
                        out_v[bag, pl.ds(t * D + c * _LANES, _LANES)] = accs[c]
                    return carry2

                lax.fori_loop(0, GROUP, bag_body, 0)
                return carry

            lax.fori_loop(0, n_groups, group_body, 0)
        pltpu.sync_copy(out_v, out.at[pl.ds(wid * bags_w, bags_w)])

    return k


def kernel(indices_t0, indices_t1, w_indices_t0, w_indices_t1,
           weights_t0, weights_t1, table0, table1, wtable0, wtable1):
    B, L = indices_t0.shape
    V, D = table0.shape
    info = plsc.get_sparse_core_info()
    k = _make_kernel(B, L, V, D, info.num_cores, info.num_subcores)
    flat_i = lambda a: a.reshape(-1).astype(jnp.int32)
    flat_f = lambda a: a.reshape(-1).astype(jnp.float32)
    return k(flat_i(indices_t0), flat_i(indices_t1),
             flat_i(w_indices_t0), flat_i(w_indices_t1),
             flat_f(weights_t0), flat_f(weights_t1),
             table0, table1, wtable0, wtable1)


# speedup vs baseline: 1.5751x; 1.5751x over previous
import functools

import jax
import jax.numpy as jnp
from jax import lax
from jax.experimental import pallas as pl
from jax.experimental.pallas import tpu as pltpu
from jax.experimental.pallas import tpu_sc as plsc

_LANES = 16
_IDX_PER_DMA = 128


@functools.lru_cache(maxsize=None)
def _make_kernel(B, L, V, D, NC, NS):
    NW = NC * NS
    bags_w = B // NW
    GROUP = 32
    idx_per_group = GROUP * L
    n_dma = idx_per_group // _IDX_PER_DMA
    n_groups = bags_w // GROUP
    CH = D // _LANES

    mesh = plsc.VectorSubcoreMesh(core_axis_name="c", subcore_axis_name="s")

    @functools.partial(
        pl.kernel,
        out_type=jax.ShapeDtypeStruct((B, 4 * D), jnp.float32),
        mesh=mesh,
        scratch_types=[
            pltpu.VMEM((bags_w * L,), jnp.int32),
            pltpu.VMEM((bags_w * L + _LANES,), jnp.float32),
            pltpu.VMEM((idx_per_group, D), jnp.float32),
            pltpu.VMEM((bags_w, 4 * D), jnp.float32),
            pltpu.SemaphoreType.DMA,
        ],
        compiler_params=pltpu.CompilerParams(use_tc_tiling_on_sc=False),
    )
    def k(i0, i1, wi0, wi1, w0, w1, t0, t1, wt0, wt1,
          out, idx_v, w_v, rows_v, out_v, sem):
        wid = lax.axis_index("s") * NC + lax.axis_index("c")
        base_idx = wid * (bags_w * L)
        tables = ((i0, None, t0, 0), (i1, None, t1, 1),
                  (wi0, w0, wt0, 2), (wi1, w1, wt1, 3))
        for ih, wh, th, t in tables:
            pltpu.sync_copy(ih.at[pl.ds(base_idx, bags_w * L)], idx_v)
            if wh is not None:
                pltpu.sync_copy(wh.at[pl.ds(base_idx, bags_w * L)],
                                w_v.at[pl.ds(0, bags_w * L)])

            def group_body(g, carry, wh=wh, th=th, t=t):
                copies = [
                    pltpu.async_copy(
                        th.at[idx_v.at[pl.ds(g * idx_per_group
                                             + kd * _IDX_PER_DMA,
                                             _IDX_PER_DMA)]],
                        rows_v.at[pl.ds(kd * _IDX_PER_DMA, _IDX_PER_DMA)],
                        sem)
                    for kd in range(n_dma)
                ]
                for cpy in copies:
                    cpy.wait()

                def bag_body(j, carry2):
                    row0 = j * L
                    bag = g * GROUP + j
                    if wh is not None:
                        wa = w_v[pl.ds(bag * L, _LANES)]
                        wb = w_v[pl.ds(bag * L + _LANES, _LANES)]
                    accs = [jnp.zeros((_LANES,), jnp.float32)
                            for _ in range(CH)]
                    for l in range(L):
                        if wh is not None:
                            src = wa if l < _LANES else wb
                            wl = jnp.take_along_axis(
                                src,
                                jnp.full((_LANES,), l % _LANES, jnp.int32),
                                axis=0)
                        for c in range(CH):
                            r = rows_v[row0 + l, pl.ds(c * _LANES, _LANES)]
                            accs[c] = accs[c] + (r * wl if wh is not None
                                                 else r)
                    for c in range(CH):
                        out_v[bag, pl.ds(t * D + c * _LANES, _LANES)] = accs[c]
                    return carry2

                lax.fori_loop(0, GROUP, bag_body, 0)
                return carry

            lax.fori_loop(0, n_groups, group_body, 0)
        pltpu.sync_copy(out_v, out.at[pl.ds(wid * bags_w, bags_w)])

    return k


def kernel(indices_t0, indices_t1, w_indices_t0, w_indices_t1,
           weights_t0, weights_t1, table0, table1, wtable0, wtable1):
    B, L = indices_t0.shape
    V, D = table0.shape
    info = plsc.get_sparse_core_info()
    k = _make_kernel(B, L, V, D, info.num_cores, info.num_subcores)
    flat_i = lambda a: a.reshape(-1).astype(jnp.int32)
    flat_f = lambda a: a.reshape(-1).astype(jnp.float32)
    return k(flat_i(indices_t0), flat_i(indices_t1),
             flat_i(w_indices_t0), flat_i(w_indices_t1),
             flat_f(weights_t0), flat_f(weights_t1),
             table0, table1, wtable0, wtable1)


# speedup vs baseline: 1.7043x; 1.0820x over previous
import functools

import jax
import jax.numpy as jnp
from jax import lax
from jax.experimental import pallas as pl
from jax.experimental.pallas import tpu as pltpu
from jax.experimental.pallas import tpu_sc as plsc

_LANES = 16
_IDX_PER_DMA = 128


@functools.lru_cache(maxsize=None)
def _make_kernel(B, L, V, D, NC, NS):
    NW = NC * NS
    bags_w = B // NW
    GROUP = 32
    idx_per_group = GROUP * L
    n_dma = idx_per_group // _IDX_PER_DMA
    n_groups = bags_w // GROUP
    CH = D // _LANES
    n_units = 4 * n_groups

    mesh = plsc.VectorSubcoreMesh(core_axis_name="c", subcore_axis_name="s")

    @functools.partial(
        pl.kernel,
        out_type=jax.ShapeDtypeStruct((B, 4 * D), jnp.float32),
        mesh=mesh,
        scratch_types=[
            pltpu.VMEM((bags_w * L,), jnp.int32),
            pltpu.VMEM((bags_w * L,), jnp.int32),
            pltpu.VMEM((bags_w * L,), jnp.int32),
            pltpu.VMEM((bags_w * L,), jnp.int32),
            pltpu.VMEM((bags_w * L + _LANES,), jnp.float32),
            pltpu.VMEM((bags_w * L + _LANES,), jnp.float32),
            pltpu.VMEM((idx_per_group, D), jnp.float32),
            pltpu.VMEM((idx_per_group, D), jnp.float32),
            pltpu.VMEM((GROUP, D), jnp.float32),
            pltpu.SemaphoreType.DMA,
            pltpu.SemaphoreType.DMA,
        ],
        compiler_params=pltpu.CompilerParams(use_tc_tiling_on_sc=False),
    )
    def k(i0, i1, wi0, wi1, w0, w1, t0, t1, wt0, wt1, out,
          idx0, idx1, idx2, idx3, wv2, wv3, rows0, rows1, outst,
          sem0, sem1):
        wid = lax.axis_index("s") * NC + lax.axis_index("c")
        base_idx = wid * (bags_w * L)
        n = bags_w * L

        for src, dst in ((i0, idx0), (i1, idx1), (wi0, idx2), (wi1, idx3)):
            pltpu.sync_copy(src.at[pl.ds(base_idx, n)], dst)
        pltpu.sync_copy(w0.at[pl.ds(base_idx, n)], wv2.at[pl.ds(0, n)])
        pltpu.sync_copy(w1.at[pl.ds(base_idx, n)], wv3.at[pl.ds(0, n)])

        tables = ((idx0, None, t0, 0), (idx1, None, t1, 1),
                  (idx2, wv2, wt0, 2), (idx3, wv3, wt1, 3))
        units = [tables[t] + (g,) for t in range(4) for g in range(n_groups)]
        rows = (rows0, rows1)
        sems = (sem0, sem1)

        def fire(u):
            idxs, _, th, _, g = units[u]
            nb = u % 2
            return [
                pltpu.async_copy(
                    th.at[idxs.at[pl.ds(g * idx_per_group + kd * _IDX_PER_DMA,
                                        _IDX_PER_DMA)]],
                    rows[nb].at[pl.ds(kd * _IDX_PER_DMA, _IDX_PER_DMA)],
                    sems[nb])
                for kd in range(n_dma)
            ]

        pending = fire(0)
        for u in range(n_units):
            nxt = fire(u + 1) if u + 1 < n_units else []
            for cpy in pending:
                cpy.wait()
            pending = nxt

            idxs, wv, th, t, g = units[u]
            rb = rows[u % 2]

            def bag_body(j, carry, wv=wv, rb=rb):
                row0 = j * L
                bag = g * GROUP + j
                if wv is not None:
                    wa = wv[pl.ds(bag * L, _LANES)]
                    wb = wv[pl.ds(bag * L + _LANES, _LANES)]
                accs = [jnp.zeros((_LANES,), jnp.float32) for _ in range(CH)]
                for l in range(L):
                    if wv is not None:
                        src = wa if l < _LANES else wb
                        wl = jnp.take_along_axis(
                            src, jnp.full((_LANES,), l % _LANES, jnp.int32),
                            axis=0)
                    for c in range(CH):
                        r = rb[row0 + l, pl.ds(c * _LANES, _LANES)]
                        accs[c] = accs[c] + (r * wl if wv is not None else r)
                for c in range(CH):
                    outst[j, pl.ds(c * _LANES, _LANES)] = accs[c]
                return carry

            lax.fori_loop(0, GROUP, bag_body, 0)
            pltpu.sync_copy(
                outst,
                out.at[pl.ds(wid * bags_w + g * GROUP, GROUP),
                       pl.ds(t * D, D)])

    return k


def kernel(indices_t0, indices_t1, w_indices_t0, w_indices_t1,
           weights_t0, weights_t1, table0, table1, wtable0, wtable1):
    B, L = indices_t0.shape
    V, D = table0.shape
    info = plsc.get_sparse_core_info()
    k = _make_kernel(B, L, V, D, info.num_cores, info.num_subcores)
    flat_i = lambda a: a.reshape(-1).astype(jnp.int32)
    flat_f = lambda a: a.reshape(-1).astype(jnp.float32)
    return k(flat_i(indices_t0), flat_i(indices_t1),
             flat_i(w_indices_t0), flat_i(w_indices_t1),
             flat_f(weights_t0), flat_f(weights_t1),
             table0, table1, wtable0, wtable1)


# speedup vs baseline: 1.7085x; 1.0025x over previous
import functools

import jax
import jax.numpy as jnp
from jax import lax
from jax.experimental import pallas as pl
from jax.experimental.pallas import tpu as pltpu
from jax.experimental.pallas import tpu_sc as plsc

_LANES = 16


@functools.lru_cache(maxsize=None)
def _make_kernel(B, L, V, D, NC, NS):
    NW = NC * NS
    bags_w = B // NW
    GROUP = 32
    rows_per_group = GROUP * L
    n_groups = bags_w // GROUP
    CH = D // _LANES
    n_units = 4 * n_groups

    mesh = plsc.VectorSubcoreMesh(core_axis_name="c", subcore_axis_name="s")

    @functools.partial(
        pl.kernel,
        out_type=jax.ShapeDtypeStruct((B, 4 * D), jnp.float32),
        mesh=mesh,
        scratch_types=[
            pltpu.VMEM((bags_w, L), jnp.int32),
            pltpu.VMEM((bags_w, L), jnp.int32),
            pltpu.VMEM((bags_w, L), jnp.int32),
            pltpu.VMEM((bags_w, L), jnp.int32),
            pltpu.VMEM((bags_w, L), jnp.float32),
            pltpu.VMEM((bags_w, L), jnp.float32),
            pltpu.VMEM((rows_per_group, D), jnp.float32),
            pltpu.VMEM((rows_per_group, D), jnp.float32),
            pltpu.VMEM((GROUP, D), jnp.float32),
            pltpu.SemaphoreType.DMA,
            pltpu.SemaphoreType.DMA,
        ],
        compiler_params=pltpu.CompilerParams(use_tc_tiling_on_sc=False),
    )
    def k(i0, i1, wi0, wi1, w0, w1, t0, t1, wt0, wt1, out,
          idx0, idx1, idx2, idx3, wv2, wv3, rows0, rows1, outst,
          sem0, sem1):
        wid = lax.axis_index("s") * NC + lax.axis_index("c")
        row0 = wid * bags_w

        for src, dst in ((i0, idx0), (i1, idx1), (wi0, idx2), (wi1, idx3),
                         (w0, wv2), (w1, wv3)):
            pltpu.sync_copy(src.at[pl.ds(row0, bags_w)], dst)

        tables = ((idx0, None, t0, 0), (idx1, None, t1, 1),
                  (idx2, wv2, wt0, 2), (idx3, wv3, wt1, 3))
        units = [tables[t] + (g,) for t in range(4) for g in range(n_groups)]
        rows = (rows0, rows1)
        sems = (sem0, sem1)

        def fire(u):
            idxs, _, th, _, g = units[u]
            nb = u % 2

            def fb(j, carry, idxs=idxs, th=th, g=g, nb=nb):
                pltpu.async_copy(th.at[idxs.at[g * GROUP + j]],
                                 rows[nb].at[pl.ds(j * L, L)], sems[nb])
                return carry

            lax.fori_loop(0, GROUP, fb, 0)

        def drain(u):
            nb = u % 2
            pltpu.make_async_copy(t0.at[pl.ds(0, rows_per_group)],
                                  rows[nb], sems[nb]).wait()

        fire(0)
        for u in range(n_units):
            if u + 1 < n_units:
                fire(u + 1)
            drain(u)

            idxs, wv, th, t, g = units[u]
            rb = rows[u % 2]

            def bag_body(j, carry, wv=wv, rb=rb, g=g):
                r0 = j * L
                bag = g * GROUP + j
                if wv is not None:
                    wa = wv[bag, pl.ds(0, _LANES)]
                    wb = wv[bag, pl.ds(L - _LANES, _LANES)]
                accs = [jnp.zeros((_LANES,), jnp.float32) for _ in range(CH)]
                for l in range(L):
                    if wv is not None:
                        if l < _LANES:
                            src_v, lane = wa, l
                        else:
                            src_v, lane = wb, l - (L - _LANES)
                        wl = jnp.take_along_axis(
                            src_v, jnp.full((_LANES,), lane, jnp.int32),
                            axis=0)
                    for c in range(CH):
                        r = rb[r0 + l, pl.ds(c * _LANES, _LANES)]
                        accs[c] = accs[c] + (r * wl if wv is not None else r)
                for c in range(CH):
                    outst[j, pl.ds(c * _LANES, _LANES)] = accs[c]
                return carry

            lax.fori_loop(0, GROUP, bag_body, 0)
            pltpu.sync_copy(
                outst,
                out.at[pl.ds(row0 + g * GROUP, GROUP), pl.ds(t * D, D)])

    return k


def kernel(indices_t0, indices_t1, w_indices_t0, w_indices_t1,
           weights_t0, weights_t1, table0, table1, wtable0, wtable1):
    B, L = indices_t0.shape
    V, D = table0.shape
    info = plsc.get_sparse_core_info()
    k = _make_kernel(B, L, V, D, info.num_cores, info.num_subcores)
    as_i32 = lambda a: a if a.dtype == jnp.int32 else a.astype(jnp.int32)
    return k(as_i32(indices_t0), as_i32(indices_t1),
             as_i32(w_indices_t0), as_i32(w_indices_t1),
             weights_t0, weights_t1,
             table0, table1, wtable0, wtable1)


# speedup vs baseline: 1.7980x; 1.0524x over previous
import functools

import jax
import jax.numpy as jnp
from jax import lax
from jax.experimental import pallas as pl
from jax.experimental.pallas import tpu as pltpu
from jax.experimental.pallas import tpu_sc as plsc

_LANES = 16


@functools.lru_cache(maxsize=None)
def _make_pack(V, D, C=1024):
    nb = -(-V // C)

    def body(a_ref, b_ref, x_ref):
        x_ref[:, 0:D] = jnp.transpose(a_ref[...])
        x_ref[:, D:2 * D] = jnp.transpose(b_ref[...])

    return pl.pallas_call(
        body,
        grid=(nb,),
        in_specs=[pl.BlockSpec((D, C), lambda i: (0, i)),
                  pl.BlockSpec((D, C), lambda i: (0, i))],
        out_specs=pl.BlockSpec((C, 2 * D), lambda i: (i, 0)),
        out_shape=jax.ShapeDtypeStruct((V, 2 * D), jnp.float32),
    )


@functools.lru_cache(maxsize=None)
def _make_bags(B, L, V, D, NC, NS, weighted):
    NW = NC * NS
    bags_w = B // NW
    GROUP = 16
    rows_per_group = GROUP * L
    n_groups = bags_w // GROUP
    CH = D // _LANES
    n_units = 2 * n_groups

    mesh = plsc.VectorSubcoreMesh(core_axis_name="c", subcore_axis_name="s")

    @functools.partial(
        pl.kernel,
        out_type=jax.ShapeDtypeStruct((B, 2 * D), jnp.float32),
        mesh=mesh,
        scratch_types=[
            pltpu.VMEM((bags_w, L), jnp.int32),
            pltpu.VMEM((bags_w, L), jnp.int32),
            pltpu.VMEM((bags_w, L), jnp.float32),
            pltpu.VMEM((bags_w, L), jnp.float32),
            pltpu.VMEM((rows_per_group, 2 * D), jnp.float32),
            pltpu.VMEM((rows_per_group, 2 * D), jnp.float32),
            pltpu.VMEM((GROUP, D), jnp.float32),
            pltpu.SemaphoreType.DMA,
            pltpu.SemaphoreType.DMA,
        ],
        compiler_params=pltpu.CompilerParams(use_tc_tiling_on_sc=False),
    )
    def k(ia, ib, *rest):
        if weighted:
            (wa, wb, x, out,
             idxa, idxb, wva, wvb, rows0, rows1, outst, sem0, sem1) = rest
        else:
            (x, out,
             idxa, idxb, wva, wvb, rows0, rows1, outst, sem0, sem1) = rest
        wid = lax.axis_index("s") * NC + lax.axis_index("c")
        row0 = wid * bags_w

        pltpu.sync_copy(ia.at[pl.ds(row0, bags_w)], idxa)
        pltpu.sync_copy(ib.at[pl.ds(row0, bags_w)], idxb)
        if weighted:
            pltpu.sync_copy(wa.at[pl.ds(row0, bags_w)], wva)
            pltpu.sync_copy(wb.at[pl.ds(row0, bags_w)], wvb)

        tables = ((idxa, wva, 0), (idxb, wvb, 1))
        units = [tables[t] + (g,) for t in range(2) for g in range(n_groups)]
        rows = (rows0, rows1)
        sems = (sem0, sem1)

        def fire(u):
            idxs, _, _, g = units[u]
            nb = u % 2

            def fb(j, carry, idxs=idxs, g=g, nb=nb):
                pltpu.async_copy(x.at[idxs.at[g * GROUP + j]],
                                 rows[nb].at[pl.ds(j * L, L)], sems[nb])
                return carry

            lax.fori_loop(0, GROUP, fb, 0)

        def drain(u):
            nb = u % 2
            pltpu.make_async_copy(x.at[pl.ds(0, rows_per_group)],
                                  rows[nb], sems[nb]).wait()

        fire(0)
        for u in range(n_units):
            if u + 1 < n_units:
                fire(u + 1)
            drain(u)

            _, wv, t, g = units[u]
            rb = rows[u % 2]
            off = t * D

            def bag_body(j, carry, wv=wv, rb=rb, g=g, off=off):
                r0 = j * L
                bag = g * GROUP + j
                if weighted:
                    w_lo = wv[bag, pl.ds(0, _LANES)]
                    w_hi = wv[bag, pl.ds(L - _LANES, _LANES)]
                accs = [jnp.zeros((_LANES,), jnp.float32) for _ in range(CH)]
                for l in range(L):
                    if weighted:
                        if l < _LANES:
                            src_v, lane = w_lo, l
                        else:
                            src_v, lane = w_hi, l - (L - _LANES)
                        wl = jnp.take_along_axis(
                            src_v, jnp.full((_LANES,), lane, jnp.int32),
                            axis=0)
                    for c in range(CH):
                        r = rb[r0 + l, pl.ds(off + c * _LANES, _LANES)]
                        accs[c] = accs[c] + (r * wl if weighted else r)
                for c in range(CH):
                    outst[j, pl.ds(c * _LANES, _LANES)] = accs[c]
                return carry

            lax.fori_loop(0, GROUP, bag_body, 0)
            pltpu.sync_copy(
                outst,
                out.at[pl.ds(row0 + g * GROUP, GROUP), pl.ds(t * D, D)])

    return k


def kernel(indices_t0, indices_t1, w_indices_t0, w_indices_t1,
           weights_t0, weights_t1, table0, table1, wtable0, wtable1):
    B, L = indices_t0.shape
    V, D = table0.shape
    info = plsc.get_sparse_core_info()
    pack = _make_pack(V, D)
    x01 = pack(jnp.transpose(table0), jnp.transpose(table1))
    xw = pack(jnp.transpose(wtable0), jnp.transpose(wtable1))
    as_i32 = lambda a: a if a.dtype == jnp.int32 else a.astype(jnp.int32)
    bags_u = _make_bags(B, L, V, D, info.num_cores, info.num_subcores, False)
    bags_w = _make_bags(B, L, V, D, info.num_cores, info.num_subcores, True)
    out01 = bags_u(as_i32(indices_t0), as_i32(indices_t1), x01)
    outw = bags_w(as_i32(w_indices_t0), as_i32(w_indices_t1),
                  weights_t0, weights_t1, xw)
    return jnp.concatenate([out01, outw], axis=1)


# speedup vs baseline: 2.1791x; 1.2120x over previous
import functools

import jax
import jax.numpy as jnp
from jax import lax
from jax.experimental import pallas as pl
from jax.experimental.pallas import tpu as pltpu
from jax.experimental.pallas import tpu_sc as plsc

_LANES = 16


@functools.lru_cache(maxsize=None)
def _make_pack(V, D, C=2048):
    nb = -(-V // C)

    def body(a_ref, b_ref, x_ref):
        eye = jnp.eye(D, dtype=jnp.float32)
        dn = (((0,), (0,)), ((), ()))
        x_ref[:, 0:D] = lax.dot_general(
            a_ref[...], eye, dn, preferred_element_type=jnp.float32)
        x_ref[:, D:2 * D] = lax.dot_general(
            b_ref[...], eye, dn, preferred_element_type=jnp.float32)

    return pl.pallas_call(
        body,
        grid=(nb,),
        in_specs=[pl.BlockSpec((D, C), lambda i: (0, i)),
                  pl.BlockSpec((D, C), lambda i: (0, i))],
        out_specs=pl.BlockSpec((C, 2 * D), lambda i: (i, 0)),
        out_shape=jax.ShapeDtypeStruct((V, 2 * D), jnp.float32),
    )


@functools.lru_cache(maxsize=None)
def _make_bags(B, L, V, D, NC, NS, weighted):
    NW = NC * NS
    bags_w = B // NW
    GROUP = 16
    rows_per_group = GROUP * L
    n_groups = bags_w // GROUP
    CH = D // _LANES
    n_units = 2 * n_groups

    mesh = plsc.VectorSubcoreMesh(core_axis_name="c", subcore_axis_name="s")

    @functools.partial(
        pl.kernel,
        out_type=jax.ShapeDtypeStruct((B, 2 * D), jnp.float32),
        mesh=mesh,
        scratch_types=[
            pltpu.VMEM((bags_w, L), jnp.int32),
            pltpu.VMEM((bags_w, L), jnp.int32),
            pltpu.VMEM((bags_w, L), jnp.float32),
            pltpu.VMEM((bags_w, L), jnp.float32),
            pltpu.VMEM((rows_per_group, 2 * D), jnp.float32),
            pltpu.VMEM((rows_per_group, 2 * D), jnp.float32),
            pltpu.VMEM((GROUP, D), jnp.float32),
            pltpu.SemaphoreType.DMA,
            pltpu.SemaphoreType.DMA,
        ],
        compiler_params=pltpu.CompilerParams(use_tc_tiling_on_sc=False),
    )
    def k(ia, ib, *rest):
        if weighted:
            (wa, wb, x, out,
             idxa, idxb, wva, wvb, rows0, rows1, outst, sem0, sem1) = rest
        else:
            (x, out,
             idxa, idxb, wva, wvb, rows0, rows1, outst, sem0, sem1) = rest
        wid = lax.axis_index("s") * NC + lax.axis_index("c")
        row0 = wid * bags_w

        pltpu.sync_copy(ia.at[pl.ds(row0, bags_w)], idxa)
        pltpu.sync_copy(ib.at[pl.ds(row0, bags_w)], idxb)
        if weighted:
            pltpu.sync_copy(wa.at[pl.ds(row0, bags_w)], wva)
            pltpu.sync_copy(wb.at[pl.ds(row0, bags_w)], wvb)

        tables = ((idxa, wva, 0), (idxb, wvb, 1))
        units = [tables[t] + (g,) for t in range(2) for g in range(n_groups)]
        rows = (rows0, rows1)
        sems = (sem0, sem1)

        def fire(u):
            idxs, _, t, g = units[u]
            nb = u % 2

            def fb(j, carry, idxs=idxs, g=g, nb=nb):
                pltpu.async_copy(x.at[idxs.at[g * GROUP + j]],
                                 rows[nb].at[pl.ds(j * L, L)], sems[nb])
                return carry

            lax.fori_loop(0, GROUP, fb, 0)

        def drain(u):
            nb = u % 2
            pltpu.make_async_copy(x.at[pl.ds(0, rows_per_group)],
                                  rows[nb], sems[nb]).wait()

        fire(0)
        for u in range(n_units):
            if u + 1 < n_units:
                fire(u + 1)
            drain(u)

            _, wv, t, g = units[u]
            rb = rows[u % 2]
            off = t * D

            def bag_body(j, carry, wv=wv, rb=rb, g=g, off=off):
                r0 = j * L
                bag = g * GROUP + j
                if weighted:
                    w_lo = wv[bag, pl.ds(0, _LANES)]
                    w_hi = wv[bag, pl.ds(L - _LANES, _LANES)]
                accs = [jnp.zeros((_LANES,), jnp.float32) for _ in range(CH)]
                for l in range(L):
                    if weighted:
                        if l < _LANES:
                            src_v, lane = w_lo, l
                        else:
                            src_v, lane = w_hi, l - (L - _LANES)
                        wl = jnp.take_along_axis(
                            src_v, jnp.full((_LANES,), lane, jnp.int32),
                            axis=0)
                    for c in range(CH):
                        r = rb[r0 + l, pl.ds(off + c * _LANES, _LANES)]
                        accs[c] = accs[c] + (r * wl if weighted else r)
                for c in range(CH):
                    outst[j, pl.ds(c * _LANES, _LANES)] = accs[c]
                return carry

            lax.fori_loop(0, GROUP, bag_body, 0)
            pltpu.sync_copy(
                outst,
                out.at[pl.ds(row0 + g * GROUP, GROUP), pl.ds(t * D, D)])

    return k


def kernel(indices_t0, indices_t1, w_indices_t0, w_indices_t1,
           weights_t0, weights_t1, table0, table1, wtable0, wtable1):
    B, L = indices_t0.shape
    V, D = table0.shape
    info = plsc.get_sparse_core_info()
    pack = _make_pack(V, D)
    x01 = pack(jnp.transpose(table0), jnp.transpose(table1))
    xw = pack(jnp.transpose(wtable0), jnp.transpose(wtable1))
    as_i32 = lambda a: a if a.dtype == jnp.int32 else a.astype(jnp.int32)
    bags_u = _make_bags(B, L, V, D, info.num_cores, info.num_subcores, False)
    bags_w = _make_bags(B, L, V, D, info.num_cores, info.num_subcores, True)
    out01 = bags_u(as_i32(indices_t0), as_i32(indices_t1), x01)
    outw = bags_w(as_i32(w_indices_t0), as_i32(w_indices_t1),
                  weights_t0, weights_t1, xw)
    return jnp.concatenate([out01, outw], axis=1)
